# Initial kernel scaffold; baseline (speedup 1.0000x reference)
import functools

import jax
import jax.numpy as jnp
from jax import lax
from jax.experimental import pallas as pl
from jax.experimental.pallas import tpu as pltpu
from jax.experimental.pallas import tpu_sc as plsc

NC = 2
NS = 16
NW = NC * NS


def _sc_gather(table, src2d, e_pad, ch):
    d = table.shape[1]
    mesh = plsc.VectorSubcoreMesh(core_axis_name="c", subcore_axis_name="s")

    @functools.partial(
        pl.kernel,
        mesh=mesh,
        out_type=jax.ShapeDtypeStruct((e_pad, d), table.dtype),
        scratch_types=[
            pltpu.VMEM((ch, 128), jnp.int32),
            pltpu.VMEM((128, d), table.dtype),
            pltpu.SemaphoreType.DMA,
        ],
    )
    def gk(table_hbm, src_hbm, out_hbm, idx_v, buf_v, sem):
        wid = lax.axis_index("s") * NC + lax.axis_index("c")
        pltpu.sync_copy(src_hbm.at[pl.ds(wid * ch, ch)], idx_v)

        @pl.loop(0, ch)
        def _(j):
            pltpu.async_copy(table_hbm.at[idx_v.at[j]], buf_v, sem).wait()
            pltpu.sync_copy(buf_v, out_hbm.at[pl.ds(wid * ch * 128 + j * 128, 128)])

    return gk(table, src2d)


def _sc_scatter_add(msg, dst2d, zeros_nk, n, ch):
    k = msg.shape[1]
    rps = n // NS
    mesh = plsc.VectorSubcoreMesh(core_axis_name="c", subcore_axis_name="s")

    @functools.partial(
        pl.kernel,
        mesh=mesh,
        out_type=jax.ShapeDtypeStruct((NC, n, k), msg.dtype),
        scratch_types=[
            pltpu.VMEM((ch, 128), jnp.int32),
            pltpu.VMEM((128, k), msg.dtype),
            pltpu.VMEM_SHARED((n, k), msg.dtype),
            pltpu.SemaphoreType.DMA,
        ],
    )
    def sk(msg_hbm, dst_hbm, zeros_hbm, out_hbm, idx_v, buf_v, acc_sh, sem):
        c = lax.axis_index("c")
        s = lax.axis_index("s")
        wid = s * NC + c
        pltpu.sync_copy(zeros_hbm.at[pl.ds(s * rps, rps)],
                        acc_sh.at[pl.ds(s * rps, rps)])
        pltpu.sync_copy(dst_hbm.at[pl.ds(wid * ch, ch)], idx_v)
        plsc.subcore_barrier()

        @pl.loop(0, ch)
        def _(j):
            pltpu.sync_copy(msg_hbm.at[pl.ds(wid * ch * 128 + j * 128, 128)], buf_v)
            pltpu.sync_copy(buf_v, acc_sh.at[idx_v.at[j]], add=True)

        plsc.subcore_barrier()
        pltpu.sync_copy(acc_sh.at[pl.ds(s * rps, rps)],
                        out_hbm.at[c, pl.ds(s * rps, rps)])

    return sk(msg, dst2d, zeros_nk)


def _msg_matmul(hsrc, ef, w4, be):
    e_pad, d_in = hsrc.shape
    ed = ef.shape[1]
    k = w4.shape[1] // ed

    def body(h_ref, ef_ref, w_ref, o_ref):
        p = jnp.dot(h_ref[...], w_ref[...], preferred_element_type=jnp.float32)
        acc = p[:, 0:k] * ef_ref[:, 0:1]
        for dd in range(1, ed):
            acc = acc + p[:, dd * k:(dd + 1) * k] * ef_ref[:, dd:dd + 1]
        o_ref[...] = acc

    return pl.pallas_call(
        body,
        grid=(e_pad // be,),
        in_specs=[
            pl.BlockSpec((be, d_in), lambda i: (i, 0)),
            pl.BlockSpec((be, ed), lambda i: (i, 0)),
            pl.BlockSpec((d_in, ed * k), lambda i: (0, 0)),
        ],
        out_specs=pl.BlockSpec((be, k), lambda i: (i, 0)),
        out_shape=jax.ShapeDtypeStruct((e_pad, k), jnp.float32),
    )(hsrc, ef, w4)


def _bn_relu(x, g, b, eps=1e-5):
    mean = jnp.mean(x, axis=0, keepdims=True)
    xc = x - mean
    var = jnp.mean(xc * xc, axis=0, keepdims=True)
    return jnp.maximum(g * xc / jnp.sqrt(var + eps) + b, 0.0)


def _tail(partials, h_self, w_self, w_neigh, gs, bs, gn, bn):
    n, k = h_self.shape[0], w_self.shape[1]

    def body(pp, hs, ws, wn, gsr, bsr, gnr, bnr, o):
        xs = jnp.dot(hs[...], ws[...], preferred_element_type=jnp.float32)
        zs = _bn_relu(xs, gsr[...], bsr[...])
        hn = pp[0] + pp[1]
        xn = jnp.dot(hn, wn[...], preferred_element_type=jnp.float32)
        zn = _bn_relu(xn, gnr[...], bnr[...])
        z = jnp.maximum(zs + zn, 0.0)
        nrm = jnp.sqrt(jnp.sum(z * z, axis=1, keepdims=True))
        nrm = jnp.where(nrm == 0.0, 1.0, nrm)
        o[...] = z / nrm

    return pl.pallas_call(
        body,
        out_shape=jax.ShapeDtypeStruct((n, k), jnp.float32),
    )(partials, h_self, w_self, w_neigh, gs, bs, gn, bn)


def kernel(h_neigh, h_self, edge_index, edge_features, W_edge, W_self, W_neigh,
           gamma_self, beta_self, gamma_neigh, beta_neigh):
    n, d_in = h_neigh.shape
    e = edge_index.shape[1]
    ed = edge_features.shape[1]
    k = W_self.shape[1]

    ch = -(-e // (NW * 128))
    e_pad = NW * ch * 128
    pad = e_pad - e
    src = jnp.concatenate([edge_index[0], jnp.zeros((pad,), jnp.int32)])
    dst = jnp.concatenate([edge_index[1], jnp.zeros((pad,), jnp.int32)])
    ef = jnp.concatenate([edge_features,
                          jnp.zeros((pad, ed), edge_features.dtype)])
    src2d = src.reshape(e_pad // 128, 128)
    dst2d = dst.reshape(e_pad // 128, 128)
    w4 = W_edge.reshape(ed, d_in, k).transpose(1, 0, 2).reshape(d_in, ed * k)

    hsrc = _sc_gather(h_neigh, src2d, e_pad, ch)
    msg = _msg_matmul(hsrc, ef, w4, 2048)
    zeros_nk = jnp.zeros((n, k), jnp.float32)
    partials = _sc_scatter_add(msg, dst2d, zeros_nk, n, ch)
    return _tail(partials, h_self, W_self, W_neigh,
                 gamma_self.reshape(1, k), beta_self.reshape(1, k),
                 gamma_neigh.reshape(1, k), beta_neigh.reshape(1, k))


# speedup vs baseline: 2.5680x; 2.5680x over previous
import functools

import jax
import jax.numpy as jnp
from jax import lax
from jax.experimental import pallas as pl
from jax.experimental.pallas import tpu as pltpu
from jax.experimental.pallas import tpu_sc as plsc

NC = 2
NS = 16
NW = NC * NS


def _sc_gather(table, src2d, e_pad, ch):
    d = table.shape[1]
    mesh = plsc.VectorSubcoreMesh(core_axis_name="c", subcore_axis_name="s")

    @functools.partial(
        pl.kernel,
        mesh=mesh,
        out_type=jax.ShapeDtypeStruct((e_pad, d), table.dtype),
        scratch_types=[
            pltpu.VMEM((ch, 128), jnp.int32),
            pltpu.VMEM((128, d), table.dtype),
            pltpu.SemaphoreType.DMA,
        ],
    )
    def gk(table_hbm, src_hbm, out_hbm, idx_v, buf_v, sem):
        wid = lax.axis_index("s") * NC + lax.axis_index("c")
        pltpu.sync_copy(src_hbm.at[pl.ds(wid * ch, ch)], idx_v)

        @pl.loop(0, ch)
        def _(j):
            pltpu.async_copy(table_hbm.at[idx_v.at[j]], buf_v, sem).wait()
            pltpu.sync_copy(buf_v, out_hbm.at[pl.ds(wid * ch * 128 + j * 128, 128)])

    return gk(table, src2d)


def _sc_scatter_add(msg, dst2d, zeros_nk, n_pad, ch):
    k = msg.shape[1]
    rps = n_pad // NS
    mesh = plsc.VectorSubcoreMesh(core_axis_name="c", subcore_axis_name="s")

    @functools.partial(
        pl.kernel,
        mesh=mesh,
        out_type=jax.ShapeDtypeStruct((NC, n_pad, k), msg.dtype),
        scratch_types=[
            pltpu.VMEM((ch, 128), jnp.int32),
            pltpu.VMEM((128, k), msg.dtype),
            pltpu.VMEM_SHARED((n_pad, k), msg.dtype),
            pltpu.SemaphoreType.DMA,
        ],
    )
    def sk(msg_hbm, dst_hbm, zeros_hbm, out_hbm, idx_v, buf_v, acc_sh, sem):
        c = lax.axis_index("c")
        s = lax.axis_index("s")
        wid = s * NC + c
        pltpu.sync_copy(zeros_hbm.at[pl.ds(s * rps, rps)],
                        acc_sh.at[pl.ds(s * rps, rps)])
        pltpu.sync_copy(dst_hbm.at[pl.ds(wid * ch, ch)], idx_v)
        plsc.subcore_barrier()

        @pl.loop(0, ch)
        def _(j):
            pltpu.sync_copy(msg_hbm.at[pl.ds(wid * ch * 128 + j * 128, 128)], buf_v)
            pltpu.sync_copy(buf_v, acc_sh.at[idx_v.at[j]], add=True)

        plsc.subcore_barrier()
        pltpu.sync_copy(acc_sh.at[pl.ds(s * rps, rps)],
                        out_hbm.at[c, pl.ds(s * rps, rps)])

    return sk(msg, dst2d, zeros_nk)


def _msg_matmul(hsrc, ef, w4, be):
    e_pad, d_in = hsrc.shape
    ed = ef.shape[1]
    k = w4.shape[1] // ed
    dsel = jnp.repeat(jnp.eye(ed, dtype=jnp.float32), k, axis=1)
    rsel = jnp.tile(jnp.eye(k, dtype=jnp.float32), (ed, 1))

    def body(h_ref, ef_ref, w_ref, s_ref, r_ref, o_ref):
        p = jnp.dot(h_ref[...], w_ref[...], preferred_element_type=jnp.float32)
        eft = jnp.dot(ef_ref[...], s_ref[...], preferred_element_type=jnp.float32)
        o_ref[...] = jnp.dot(p * eft, r_ref[...],
                             preferred_element_type=jnp.float32)

    return pl.pallas_call(
        body,
        grid=(e_pad // be,),
        in_specs=[
            pl.BlockSpec((be, d_in), lambda i: (i, 0)),
            pl.BlockSpec((be, ed), lambda i: (i, 0)),
            pl.BlockSpec((d_in, ed * k), lambda i: (0, 0)),
            pl.BlockSpec((ed, ed * k), lambda i: (0, 0)),
            pl.BlockSpec((ed * k, k), lambda i: (0, 0)),
        ],
        out_specs=pl.BlockSpec((be, k), lambda i: (i, 0)),
        out_shape=jax.ShapeDtypeStruct((e_pad, k), jnp.float32),
    )(hsrc, ef, w4, dsel, rsel)


def _bn_relu(x, g, b, eps=1e-5):
    mean = jnp.mean(x, axis=0, keepdims=True)
    xc = x - mean
    var = jnp.mean(xc * xc, axis=0, keepdims=True)
    return jnp.maximum(g * xc / jnp.sqrt(var + eps) + b, 0.0)


def _tail(partials, h_self, w_self, w_neigh, gs, bs, gn, bn):
    n, k = h_self.shape[0], w_self.shape[1]

    def body(pp, hs, ws, wn, gsr, bsr, gnr, bnr, o):
        xs = jnp.dot(hs[...], ws[...], preferred_element_type=jnp.float32)
        zs = _bn_relu(xs, gsr[...], bsr[...])
        hn = pp[0] + pp[1]
        xn = jnp.dot(hn, wn[...], preferred_element_type=jnp.float32)
        zn = _bn_relu(xn, gnr[...], bnr[...])
        z = jnp.maximum(zs + zn, 0.0)
        nrm = jnp.sqrt(jnp.sum(z * z, axis=1, keepdims=True))
        nrm = jnp.where(nrm == 0.0, 1.0, nrm)
        o[...] = z / nrm

    return pl.pallas_call(
        body,
        out_shape=jax.ShapeDtypeStruct((n, k), jnp.float32),
    )(partials, h_self, w_self, w_neigh, gs, bs, gn, bn)


def kernel(h_neigh, h_self, edge_index, edge_features, W_edge, W_self, W_neigh,
           gamma_self, beta_self, gamma_neigh, beta_neigh):
    n, d_in = h_neigh.shape
    e = edge_index.shape[1]
    ed = edge_features.shape[1]
    k = W_self.shape[1]

    ch = -(-e // (NW * 128))
    e_pad = NW * ch * 128
    pad = e_pad - e
    src = jnp.concatenate([edge_index[0], jnp.zeros((pad,), jnp.int32)])
    dst = jnp.concatenate([edge_index[1], jnp.zeros((pad,), jnp.int32)])
    ef = jnp.concatenate([edge_features,
                          jnp.zeros((pad, ed), edge_features.dtype)])
    src2d = src.reshape(e_pad // 128, 128)
    dst2d = dst.reshape(e_pad // 128, 128)
    w4 = W_edge.reshape(ed, d_in, k).transpose(1, 0, 2).reshape(d_in, ed * k)

    hsrc = _sc_gather(h_neigh, src2d, e_pad, ch)
    msg = _msg_matmul(hsrc, ef, w4, 2048)
    n_pad = -(-n // (NS * 8)) * NS * 8
    zeros_nk = jnp.zeros((n_pad, k), jnp.float32)
    partials = _sc_scatter_add(msg, dst2d, zeros_nk, n_pad, ch)
    partials = partials[:, :n, :]
    return _tail(partials, h_self, W_self, W_neigh,
                 gamma_self.reshape(1, k), beta_self.reshape(1, k),
                 gamma_neigh.reshape(1, k), beta_neigh.reshape(1, k))
